# Initial kernel scaffold; baseline (speedup 1.0000x reference)
import functools

import jax
import jax.numpy as jnp
from jax import lax
from jax.experimental import pallas as pl
from jax.experimental.pallas import tpu as pltpu
from jax.experimental.pallas import tpu_sc as plsc

_N = 10000
_E = 320000
_D = 128

_NC = 2
_NS = 16
_NW = _NC * _NS
_EPT = _E // _NW
_BLK = 125
_NB = _EPT // _BLK
_RPT = _N // _NS
_DEGW = 16
_EPS = 1e-5


def _make_sc_agg(with_deg: bool):
    mesh = plsc.VectorSubcoreMesh(core_axis_name="c", subcore_axis_name="s")

    out_type = [jax.ShapeDtypeStruct((_NC * _N, _D), jnp.float32)]
    scratch = [
        pltpu.VMEM((_NB, _BLK), jnp.int32),
        pltpu.VMEM((_NB, _BLK), jnp.int32),
        pltpu.VMEM((_BLK, _D), jnp.float32),
        pltpu.VMEM((_BLK, _D), jnp.float32),
        pltpu.VMEM_SHARED((_N, _D), jnp.float32),
        pltpu.SemaphoreType.DMA,
        pltpu.SemaphoreType.DMA,
    ]
    if with_deg:
        out_type.append(jax.ShapeDtypeStruct((_NC * _N, _DEGW), jnp.float32))
        scratch.append(pltpu.VMEM((_BLK, _DEGW), jnp.float32))
        scratch.append(pltpu.VMEM_SHARED((_N, _DEGW), jnp.float32))

    def body(*args):
        if with_deg:
            (h_hbm, src_hbm, dst_hbm, zrow_hbm, zdeg_hbm, one_hbm,
             agg_out, deg_out,
             src_v, dst_v, rowa, rowb, agg_sh, sema, semb, one_v, deg_sh) = args
        else:
            (h_hbm, src_hbm, dst_hbm, zrow_hbm,
             agg_out,
             src_v, dst_v, rowa, rowb, agg_sh, sema, semb) = args

        c = lax.axis_index("c")
        s = lax.axis_index("s")
        wid = c * _NS + s

        pltpu.sync_copy(src_hbm.at[wid], src_v)
        pltpu.sync_copy(dst_hbm.at[wid], dst_v)
        pltpu.sync_copy(zrow_hbm, agg_sh.at[pl.ds(s * _RPT, _RPT)])
        if with_deg:
            pltpu.sync_copy(zdeg_hbm, deg_sh.at[pl.ds(s * _RPT, _RPT)])
            pltpu.sync_copy(one_hbm, one_v)

        pltpu.async_copy(h_hbm.at[src_v.at[0]], rowa, sema)
        plsc.subcore_barrier()

        def step(i, carry):
            j0 = 2 * i
            for b in range(2):
                j = j0 + b
                buf, sem = (rowa, sema) if b == 0 else (rowb, semb)
                nbuf, nsem = (rowb, semb) if b == 0 else (rowa, sema)
                nxt = j + 1

                @pl.when(nxt < _NB)
                def _():
                    pltpu.async_copy(h_hbm.at[src_v.at[nxt]], nbuf, nsem)

                pltpu.make_async_copy(h_hbm.at[src_v.at[j]], buf, sem).wait()
                pltpu.sync_copy(buf, agg_sh.at[dst_v.at[j]], add=True)
                if with_deg:
                    pltpu.sync_copy(one_v, deg_sh.at[dst_v.at[j]], add=True)
            return carry

        lax.fori_loop(0, _NB // 2, step, 0)

        plsc.subcore_barrier()
        base = c * _N + s * _RPT
        pltpu.sync_copy(agg_sh.at[pl.ds(s * _RPT, _RPT)],
                        agg_out.at[pl.ds(base, _RPT)])
        if with_deg:
            pltpu.sync_copy(deg_sh.at[pl.ds(s * _RPT, _RPT)],
                            deg_out.at[pl.ds(base, _RPT)])

    return pl.kernel(
        body,
        out_type=tuple(out_type) if with_deg else out_type[0],
        mesh=mesh,
        scratch_types=scratch,
    )


_sc_agg_deg = _make_sc_agg(True)
_sc_agg = _make_sc_agg(False)


def _tc_layer0(p, degp, x, Wl, Wr, b, g, be):
    def body(p_ref, degp_ref, x_ref, wl_ref, wr_ref, b_ref, g_ref, be_ref,
             h_ref, dinv_ref):
        dp = degp_ref[...]
        deg = dp[0, :, :1] + dp[1, :, :1]
        dinv = 1.0 / jnp.maximum(deg, 1.0)
        dinv_ref[...] = dinv
        agg = p_ref[0] + p_ref[1]
        mean = agg * dinv
        h = (jnp.dot(mean, wl_ref[...], preferred_element_type=jnp.float32)
             + jnp.dot(x_ref[...], wr_ref[...], preferred_element_type=jnp.float32)
             + b_ref[...])
        m = jnp.mean(h, axis=0, keepdims=True)
        hc = h - m
        v = jnp.mean(hc * hc, axis=0, keepdims=True)
        scale = g_ref[...] * lax.rsqrt(v + _EPS)
        h_ref[...] = jnp.maximum(hc * scale + be_ref[...], 0.0)

    return pl.pallas_call(
        body,
        out_shape=(jax.ShapeDtypeStruct((_N, _D), jnp.float32),
                   jax.ShapeDtypeStruct((_N, 1), jnp.float32)),
    )(p, degp, x, Wl, Wr, b, g, be)


def _tc_layer_mid(p, dinv, x, Wl, Wr, b, g, be):
    def body(p_ref, dinv_ref, x_ref, wl_ref, wr_ref, b_ref, g_ref, be_ref,
             h_ref):
        mean = (p_ref[0] + p_ref[1]) * dinv_ref[...]
        h = (jnp.dot(mean, wl_ref[...], preferred_element_type=jnp.float32)
             + jnp.dot(x_ref[...], wr_ref[...], preferred_element_type=jnp.float32)
             + b_ref[...])
        m = jnp.mean(h, axis=0, keepdims=True)
        hc = h - m
        v = jnp.mean(hc * hc, axis=0, keepdims=True)
        scale = g_ref[...] * lax.rsqrt(v + _EPS)
        h_ref[...] = jnp.maximum(hc * scale + be_ref[...], 0.0)

    return pl.pallas_call(
        body,
        out_shape=jax.ShapeDtypeStruct((_N, _D), jnp.float32),
    )(p, dinv, x, Wl, Wr, b, g, be)


def _tc_layer_last(p, dinv, x, Wl, Wr, b):
    def body(p_ref, dinv_ref, x_ref, wl_ref, wr_ref, b_ref, h_ref):
        mean = (p_ref[0] + p_ref[1]) * dinv_ref[...]
        h_ref[...] = (
            jnp.dot(mean, wl_ref[...], preferred_element_type=jnp.float32)
            + jnp.dot(x_ref[...], wr_ref[...], preferred_element_type=jnp.float32)
            + b_ref[...])

    return pl.pallas_call(
        body,
        out_shape=jax.ShapeDtypeStruct((_N, _D), jnp.float32),
    )(p, dinv, x, Wl, Wr, b)


def kernel(x, edge_index, Wl0, Wr0, b0, Wl1, Wr1, b1, Wl2, Wr2, b2,
           g0, be0, g1, be1):
    src = edge_index[0].astype(jnp.int32).reshape(_NW, _NB, _BLK)
    dst = edge_index[1].astype(jnp.int32).reshape(_NW, _NB, _BLK)
    zrow = jnp.zeros((_RPT, _D), jnp.float32)
    zdeg = jnp.zeros((_RPT, _DEGW), jnp.float32)
    ones = jnp.ones((_BLK, _DEGW), jnp.float32)

    b0r, b1r, b2r = b0.reshape(1, _D), b1.reshape(1, _D), b2.reshape(1, _D)
    g0r, g1r = g0.reshape(1, _D), g1.reshape(1, _D)
    be0r, be1r = be0.reshape(1, _D), be1.reshape(1, _D)

    p0, degp = _sc_agg_deg(x, src, dst, zrow, zdeg, ones)
    h0, dinv = _tc_layer0(p0.reshape(_NC, _N, _D),
                          degp.reshape(_NC, _N, _DEGW),
                          x, Wl0, Wr0, b0r, g0r, be0r)
    p1 = _sc_agg(h0, src, dst, zrow)
    h1 = _tc_layer_mid(p1.reshape(_NC, _N, _D), dinv, h0,
                       Wl1, Wr1, b1r, g1r, be1r)
    p2 = _sc_agg(h1, src, dst, zrow)
    return _tc_layer_last(p2.reshape(_NC, _N, _D), dinv, h1, Wl2, Wr2, b2r)


# speedup vs baseline: 4.3413x; 4.3413x over previous
import jax
import jax.numpy as jnp
from jax import lax
from jax.experimental import pallas as pl
from jax.experimental.pallas import tpu as pltpu
from jax.experimental.pallas import tpu_sc as plsc

_N = 10000
_E = 320000
_D = 128

_NC = 2
_NS = 16
_NW = _NC * _NS
_EPT = _E // _NW
_BLK = 80
_NB = _EPT // _BLK
_NPAD = 10240
_RPT = _NPAD // _NS
_EPS = 1e-5

_sc_mesh = plsc.VectorSubcoreMesh(core_axis_name="c", subcore_axis_name="s")


def _sc_agg_body(h_hbm, src_hbm, dst_hbm, zrow_hbm,
                 agg_out, src_v, dst_v, rowv, agg_sh):
    c = lax.axis_index("c")
    s = lax.axis_index("s")
    wid = c * _NS + s
    ebase = wid * _EPT

    pltpu.sync_copy(zrow_hbm, agg_sh.at[pl.ds(s * _RPT, _RPT)])
    plsc.subcore_barrier()

    def step(j, carry):
        pltpu.sync_copy(src_hbm.at[pl.ds(ebase + j * _BLK, _BLK)], src_v)
        pltpu.sync_copy(dst_hbm.at[pl.ds(ebase + j * _BLK, _BLK)], dst_v)
        pltpu.sync_copy(h_hbm.at[src_v], rowv)
        pltpu.sync_copy(rowv, agg_sh.at[dst_v], add=True)
        return carry

    lax.fori_loop(0, _NB, step, 0)

    plsc.subcore_barrier()
    base = c * _NPAD + s * _RPT
    pltpu.sync_copy(agg_sh.at[pl.ds(s * _RPT, _RPT)],
                    agg_out.at[pl.ds(base, _RPT)])


_sc_agg = pl.kernel(
    _sc_agg_body,
    out_type=jax.ShapeDtypeStruct((_NC * _NPAD, _D), jnp.float32),
    mesh=_sc_mesh,
    scratch_types=[
        pltpu.VMEM((_BLK,), jnp.int32),
        pltpu.VMEM((_BLK,), jnp.int32),
        pltpu.VMEM((_BLK, _D), jnp.float32),
        pltpu.VMEM_SHARED((_NPAD, _D), jnp.float32),
    ],
)


def _sc_deg_body(dst_hbm, zrow_hbm, one_hbm,
                 deg_out, dst_v, one_v, deg_sh):
    c = lax.axis_index("c")
    s = lax.axis_index("s")
    wid = c * _NS + s
    ebase = wid * _EPT

    pltpu.sync_copy(zrow_hbm, deg_sh.at[pl.ds(s * _RPT, _RPT)])
    pltpu.sync_copy(one_hbm, one_v)
    plsc.subcore_barrier()

    def step(j, carry):
        pltpu.sync_copy(dst_hbm.at[pl.ds(ebase + j * _BLK, _BLK)], dst_v)
        pltpu.sync_copy(one_v, deg_sh.at[dst_v], add=True)
        return carry

    lax.fori_loop(0, _NB, step, 0)

    plsc.subcore_barrier()
    base = c * _NPAD + s * _RPT
    pltpu.sync_copy(deg_sh.at[pl.ds(s * _RPT, _RPT)],
                    deg_out.at[pl.ds(base, _RPT)])


_sc_deg = pl.kernel(
    _sc_deg_body,
    out_type=jax.ShapeDtypeStruct((_NC * _NPAD, _D), jnp.float32),
    mesh=_sc_mesh,
    scratch_types=[
        pltpu.VMEM((_BLK,), jnp.int32),
        pltpu.VMEM((_BLK, _D), jnp.float32),
        pltpu.VMEM_SHARED((_NPAD, _D), jnp.float32),
    ],
)


def _tc_layer0(p, degp, x, Wl, Wr, b, g, be):
    def body(p_ref, degp_ref, x_ref, wl_ref, wr_ref, b_ref, g_ref, be_ref,
             h_ref, dinv_ref):
        deg = degp_ref[0, :_N] + degp_ref[1, :_N]
        dinv = 1.0 / jnp.maximum(deg, 1.0)
        dinv_ref[...] = dinv
        agg = p_ref[0, :_N] + p_ref[1, :_N]
        mean = agg * dinv
        h = (jnp.dot(mean, wl_ref[...], preferred_element_type=jnp.float32)
             + jnp.dot(x_ref[...], wr_ref[...], preferred_element_type=jnp.float32)
             + b_ref[...])
        m = jnp.mean(h, axis=0, keepdims=True)
        hc = h - m
        v = jnp.mean(hc * hc, axis=0, keepdims=True)
        scale = g_ref[...] * lax.rsqrt(v + _EPS)
        h_ref[...] = jnp.maximum(hc * scale + be_ref[...], 0.0)

    return pl.pallas_call(
        body,
        out_shape=(jax.ShapeDtypeStruct((_N, _D), jnp.float32),
                   jax.ShapeDtypeStruct((_N, _D), jnp.float32)),
    )(p, degp, x, Wl, Wr, b, g, be)


def _tc_layer_mid(p, dinv, x, Wl, Wr, b, g, be):
    def body(p_ref, dinv_ref, x_ref, wl_ref, wr_ref, b_ref, g_ref, be_ref,
             h_ref):
        mean = (p_ref[0, :_N] + p_ref[1, :_N]) * dinv_ref[...]
        h = (jnp.dot(mean, wl_ref[...], preferred_element_type=jnp.float32)
             + jnp.dot(x_ref[...], wr_ref[...], preferred_element_type=jnp.float32)
             + b_ref[...])
        m = jnp.mean(h, axis=0, keepdims=True)
        hc = h - m
        v = jnp.mean(hc * hc, axis=0, keepdims=True)
        scale = g_ref[...] * lax.rsqrt(v + _EPS)
        h_ref[...] = jnp.maximum(hc * scale + be_ref[...], 0.0)

    return pl.pallas_call(
        body,
        out_shape=jax.ShapeDtypeStruct((_N, _D), jnp.float32),
    )(p, dinv, x, Wl, Wr, b, g, be)


def _tc_layer_last(p, dinv, x, Wl, Wr, b):
    def body(p_ref, dinv_ref, x_ref, wl_ref, wr_ref, b_ref, h_ref):
        mean = (p_ref[0, :_N] + p_ref[1, :_N]) * dinv_ref[...]
        h_ref[...] = (
            jnp.dot(mean, wl_ref[...], preferred_element_type=jnp.float32)
            + jnp.dot(x_ref[...], wr_ref[...], preferred_element_type=jnp.float32)
            + b_ref[...])

    return pl.pallas_call(
        body,
        out_shape=jax.ShapeDtypeStruct((_N, _D), jnp.float32),
    )(p, dinv, x, Wl, Wr, b)


def kernel(x, edge_index, Wl0, Wr0, b0, Wl1, Wr1, b1, Wl2, Wr2, b2,
           g0, be0, g1, be1):
    src = edge_index[0].astype(jnp.int32)
    dst = edge_index[1].astype(jnp.int32)
    zrow = jnp.zeros((_RPT, _D), jnp.float32)
    ones = jnp.ones((_BLK, _D), jnp.float32)

    b0r, b1r, b2r = b0.reshape(1, _D), b1.reshape(1, _D), b2.reshape(1, _D)
    g0r, g1r = g0.reshape(1, _D), g1.reshape(1, _D)
    be0r, be1r = be0.reshape(1, _D), be1.reshape(1, _D)

    degp = _sc_deg(dst, zrow, ones)
    p0 = _sc_agg(x, src, dst, zrow)
    h0, dinv = _tc_layer0(p0.reshape(_NC, _NPAD, _D),
                          degp.reshape(_NC, _NPAD, _D),
                          x, Wl0, Wr0, b0r, g0r, be0r)
    p1 = _sc_agg(h0, src, dst, zrow)
    h1 = _tc_layer_mid(p1.reshape(_NC, _NPAD, _D), dinv, h0,
                       Wl1, Wr1, b1r, g1r, be1r)
    p2 = _sc_agg(h1, src, dst, zrow)
    return _tc_layer_last(p2.reshape(_NC, _NPAD, _D), dinv, h1, Wl2, Wr2, b2r)


# speedup vs baseline: 9.7117x; 2.2371x over previous
import jax
import jax.numpy as jnp
from jax import lax
from jax.experimental import pallas as pl
from jax.experimental.pallas import tpu as pltpu
from jax.experimental.pallas import tpu_sc as plsc

_N = 10000
_E = 320000
_D = 128

_NC = 2
_NS = 16
_NW = _NC * _NS
_EPT = _E // _NW
_BLK = 80
_NB = _EPT // _BLK
_NPAD = 10240
_RPT = _NPAD // _NS
_EPS = 1e-5

_sc_mesh = plsc.VectorSubcoreMesh(core_axis_name="c", subcore_axis_name="s")


def _sc_agg_body(h_hbm, src_hbm, dst_hbm, zrow_hbm,
                 agg_out, src_v, dst0, dst1, rowa, rowb, agg_sh,
                 sga, sgb, sia, sib):
    c = lax.axis_index("c")
    s = lax.axis_index("s")
    wid = c * _NS + s
    ebase = wid * _EPT
    dsts = (dst0, dst1)
    rows = (rowa, rowb)
    sg = (sga, sgb)
    si = (sia, sib)

    pltpu.sync_copy(zrow_hbm, agg_sh.at[pl.ds(s * _RPT, _RPT)])
    pltpu.sync_copy(src_hbm.at[pl.ds(ebase, _EPT)], src_v)
    pltpu.async_copy(dst_hbm.at[pl.ds(ebase, _BLK)], dst0, sia)
    pltpu.async_copy(h_hbm.at[src_v.at[pl.ds(0, _BLK)]], rowa, sga)
    plsc.subcore_barrier()

    def do_block(j, b):
        nb = 1 - b
        @pl.when(j + 1 < _NB)
        def _():
            pltpu.async_copy(
                dst_hbm.at[pl.ds(ebase + (j + 1) * _BLK, _BLK)],
                dsts[nb], si[nb])
            pltpu.async_copy(
                h_hbm.at[src_v.at[pl.ds((j + 1) * _BLK, _BLK)]],
                rows[nb], sg[nb])

        pltpu.make_async_copy(
            h_hbm.at[src_v.at[pl.ds(j * _BLK, _BLK)]], rows[b], sg[b]).wait()
        pltpu.make_async_copy(
            dst_hbm.at[pl.ds(ebase + j * _BLK, _BLK)], dsts[b], si[b]).wait()
        pltpu.sync_copy(rows[b], agg_sh.at[dsts[b]], add=True)

    def step(i, carry):
        do_block(2 * i, 0)
        do_block(2 * i + 1, 1)
        return carry

    lax.fori_loop(0, _NB // 2, step, 0)
    if _NB % 2:
        do_block(_NB - 1, 0)

    plsc.subcore_barrier()
    base = c * _NPAD + s * _RPT
    pltpu.sync_copy(agg_sh.at[pl.ds(s * _RPT, _RPT)],
                    agg_out.at[pl.ds(base, _RPT)])


_sc_agg = pl.kernel(
    _sc_agg_body,
    out_type=jax.ShapeDtypeStruct((_NC * _NPAD, _D), jnp.float32),
    mesh=_sc_mesh,
    scratch_types=[
        pltpu.VMEM((_EPT,), jnp.int32),
        pltpu.VMEM((_BLK,), jnp.int32),
        pltpu.VMEM((_BLK,), jnp.int32),
        pltpu.VMEM((_BLK, _D), jnp.float32),
        pltpu.VMEM((_BLK, _D), jnp.float32),
        pltpu.VMEM_SHARED((_NPAD, _D), jnp.float32),
        pltpu.SemaphoreType.DMA,
        pltpu.SemaphoreType.DMA,
        pltpu.SemaphoreType.DMA,
        pltpu.SemaphoreType.DMA,
    ],
)


def _sc_deg_body(dst_hbm, zrow_hbm, one_hbm,
                 deg_out, dst0, dst1, one_v, deg_sh, sia, sib):
    c = lax.axis_index("c")
    s = lax.axis_index("s")
    wid = c * _NS + s
    ebase = wid * _EPT
    dsts = (dst0, dst1)
    si = (sia, sib)

    pltpu.sync_copy(zrow_hbm, deg_sh.at[pl.ds(s * _RPT, _RPT)])
    pltpu.sync_copy(one_hbm, one_v)
    pltpu.async_copy(dst_hbm.at[pl.ds(ebase, _BLK)], dst0, sia)
    plsc.subcore_barrier()

    def do_block(j, b):
        nb = 1 - b
        @pl.when(j + 1 < _NB)
        def _():
            pltpu.async_copy(
                dst_hbm.at[pl.ds(ebase + (j + 1) * _BLK, _BLK)],
                dsts[nb], si[nb])

        pltpu.make_async_copy(
            dst_hbm.at[pl.ds(ebase + j * _BLK, _BLK)], dsts[b], si[b]).wait()
        pltpu.sync_copy(one_v, deg_sh.at[dsts[b]], add=True)

    def step(i, carry):
        do_block(2 * i, 0)
        do_block(2 * i + 1, 1)
        return carry

    lax.fori_loop(0, _NB // 2, step, 0)
    if _NB % 2:
        do_block(_NB - 1, 0)

    plsc.subcore_barrier()
    base = c * _NPAD + s * _RPT
    pltpu.sync_copy(deg_sh.at[pl.ds(s * _RPT, _RPT)],
                    deg_out.at[pl.ds(base, _RPT)])


_sc_deg = pl.kernel(
    _sc_deg_body,
    out_type=jax.ShapeDtypeStruct((_NC * _NPAD, _D), jnp.float32),
    mesh=_sc_mesh,
    scratch_types=[
        pltpu.VMEM((_BLK,), jnp.int32),
        pltpu.VMEM((_BLK,), jnp.int32),
        pltpu.VMEM((_BLK, _D), jnp.float32),
        pltpu.VMEM_SHARED((_NPAD, _D), jnp.float32),
        pltpu.SemaphoreType.DMA,
        pltpu.SemaphoreType.DMA,
    ],
)


def _tc_layer0(p, degp, x, Wl, Wr, b, g, be):
    def body(p_ref, degp_ref, x_ref, wl_ref, wr_ref, b_ref, g_ref, be_ref,
             h_ref, dinv_ref):
        deg = degp_ref[0, :_N] + degp_ref[1, :_N]
        dinv = 1.0 / jnp.maximum(deg, 1.0)
        dinv_ref[...] = dinv
        agg = p_ref[0, :_N] + p_ref[1, :_N]
        mean = agg * dinv
        h = (jnp.dot(mean, wl_ref[...], preferred_element_type=jnp.float32)
             + jnp.dot(x_ref[...], wr_ref[...], preferred_element_type=jnp.float32)
             + b_ref[...])
        m = jnp.mean(h, axis=0, keepdims=True)
        hc = h - m
        v = jnp.mean(hc * hc, axis=0, keepdims=True)
        scale = g_ref[...] * lax.rsqrt(v + _EPS)
        h_ref[...] = jnp.maximum(hc * scale + be_ref[...], 0.0)

    return pl.pallas_call(
        body,
        out_shape=(jax.ShapeDtypeStruct((_N, _D), jnp.float32),
                   jax.ShapeDtypeStruct((_N, _D), jnp.float32)),
    )(p, degp, x, Wl, Wr, b, g, be)


def _tc_layer_mid(p, dinv, x, Wl, Wr, b, g, be):
    def body(p_ref, dinv_ref, x_ref, wl_ref, wr_ref, b_ref, g_ref, be_ref,
             h_ref):
        mean = (p_ref[0, :_N] + p_ref[1, :_N]) * dinv_ref[...]
        h = (jnp.dot(mean, wl_ref[...], preferred_element_type=jnp.float32)
             + jnp.dot(x_ref[...], wr_ref[...], preferred_element_type=jnp.float32)
             + b_ref[...])
        m = jnp.mean(h, axis=0, keepdims=True)
        hc = h - m
        v = jnp.mean(hc * hc, axis=0, keepdims=True)
        scale = g_ref[...] * lax.rsqrt(v + _EPS)
        h_ref[...] = jnp.maximum(hc * scale + be_ref[...], 0.0)

    return pl.pallas_call(
        body,
        out_shape=jax.ShapeDtypeStruct((_N, _D), jnp.float32),
    )(p, dinv, x, Wl, Wr, b, g, be)


def _tc_layer_last(p, dinv, x, Wl, Wr, b):
    def body(p_ref, dinv_ref, x_ref, wl_ref, wr_ref, b_ref, h_ref):
        mean = (p_ref[0, :_N] + p_ref[1, :_N]) * dinv_ref[...]
        h_ref[...] = (
            jnp.dot(mean, wl_ref[...], preferred_element_type=jnp.float32)
            + jnp.dot(x_ref[...], wr_ref[...], preferred_element_type=jnp.float32)
            + b_ref[...])

    return pl.pallas_call(
        body,
        out_shape=jax.ShapeDtypeStruct((_N, _D), jnp.float32),
    )(p, dinv, x, Wl, Wr, b)


def kernel(x, edge_index, Wl0, Wr0, b0, Wl1, Wr1, b1, Wl2, Wr2, b2,
           g0, be0, g1, be1):
    src = edge_index[0].astype(jnp.int32)
    dst = edge_index[1].astype(jnp.int32)
    zrow = jnp.zeros((_RPT, _D), jnp.float32)
    ones = jnp.ones((_BLK, _D), jnp.float32)

    b0r, b1r, b2r = b0.reshape(1, _D), b1.reshape(1, _D), b2.reshape(1, _D)
    g0r, g1r = g0.reshape(1, _D), g1.reshape(1, _D)
    be0r, be1r = be0.reshape(1, _D), be1.reshape(1, _D)

    degp = _sc_deg(dst, zrow, ones)
    p0 = _sc_agg(x, src, dst, zrow)
    h0, dinv = _tc_layer0(p0.reshape(_NC, _NPAD, _D),
                          degp.reshape(_NC, _NPAD, _D),
                          x, Wl0, Wr0, b0r, g0r, be0r)
    p1 = _sc_agg(h0, src, dst, zrow)
    h1 = _tc_layer_mid(p1.reshape(_NC, _NPAD, _D), dinv, h0,
                       Wl1, Wr1, b1r, g1r, be1r)
    p2 = _sc_agg(h1, src, dst, zrow)
    return _tc_layer_last(p2.reshape(_NC, _NPAD, _D), dinv, h1, Wl2, Wr2, b2r)


# speedup vs baseline: 11.2414x; 1.1575x over previous
import jax
import jax.numpy as jnp
from jax import lax
from jax.experimental import pallas as pl
from jax.experimental.pallas import tpu as pltpu
from jax.experimental.pallas import tpu_sc as plsc

_N = 10000
_E = 320000
_D = 128

_NC = 2
_NS = 16
_NW = _NC * _NS
_EPT = _E // _NW
_BLK = 80
_NB = _EPT // _BLK
_NPAD = 10240
_RPT = _NPAD // _NS
_EPS = 1e-5

_sc_mesh = plsc.VectorSubcoreMesh(core_axis_name="c", subcore_axis_name="s")


_NSLOT = 4


def _sc_agg_body(h_hbm, src_hbm, dst_hbm, zrow_hbm,
                 agg_out, *sc):
    srcs = sc[0:4]
    dsts = sc[4:8]
    rows = sc[8:12]
    agg_sh = sc[12]
    ssem = sc[13:17]
    dsem = sc[17:21]
    gsem = sc[21:25]
    c = lax.axis_index("c")
    s = lax.axis_index("s")
    wid = c * _NS + s
    ebase = wid * _EPT

    def launch_idx(j, b):
        pltpu.async_copy(src_hbm.at[pl.ds(ebase + j * _BLK, _BLK)],
                         srcs[b], ssem[b])
        pltpu.async_copy(dst_hbm.at[pl.ds(ebase + j * _BLK, _BLK)],
                         dsts[b], dsem[b])

    def launch_gather(j, b):
        pltpu.make_async_copy(src_hbm.at[pl.ds(ebase + j * _BLK, _BLK)],
                              srcs[b], ssem[b]).wait()
        pltpu.make_async_copy(dst_hbm.at[pl.ds(ebase + j * _BLK, _BLK)],
                              dsts[b], dsem[b]).wait()
        pltpu.async_copy(h_hbm.at[srcs[b]], rows[b], gsem[b])

    def do_scatter(j, b):
        pltpu.make_async_copy(h_hbm.at[srcs[b]], rows[b], gsem[b]).wait()
        pltpu.sync_copy(rows[b], agg_sh.at[dsts[b]], add=True)

    pltpu.sync_copy(zrow_hbm, agg_sh.at[pl.ds(s * _RPT, _RPT)])
    launch_idx(0, 0)
    launch_idx(1, 1)
    launch_idx(2, 2)
    launch_gather(0, 0)
    launch_gather(1, 1)
    plsc.subcore_barrier()

    def do_block(j, k):
        @pl.when(j + 3 < _NB)
        def _():
            launch_idx(j + 3, (k + 3) % _NSLOT)

        @pl.when(j + 2 < _NB)
        def _():
            launch_gather(j + 2, (k + 2) % _NSLOT)

        do_scatter(j, k)

    def step(i, carry):
        for k in range(_NSLOT):
            do_block(_NSLOT * i + k, k)
        return carry

    lax.fori_loop(0, _NB // _NSLOT, step, 0)
    for k in range(_NB % _NSLOT):
        do_block((_NB // _NSLOT) * _NSLOT + k, k)

    plsc.subcore_barrier()
    base = c * _NPAD + s * _RPT
    pltpu.sync_copy(agg_sh.at[pl.ds(s * _RPT, _RPT)],
                    agg_out.at[pl.ds(base, _RPT)])


_sc_agg = pl.kernel(
    _sc_agg_body,
    out_type=jax.ShapeDtypeStruct((_NC * _NPAD, _D), jnp.float32),
    mesh=_sc_mesh,
    scratch_types=(
        [pltpu.VMEM((_BLK,), jnp.int32) for _ in range(_NSLOT)]
        + [pltpu.VMEM((_BLK,), jnp.int32) for _ in range(_NSLOT)]
        + [pltpu.VMEM((_BLK, _D), jnp.float32) for _ in range(_NSLOT)]
        + [pltpu.VMEM_SHARED((_NPAD, _D), jnp.float32)]
        + [pltpu.SemaphoreType.DMA] * (3 * _NSLOT)
    ),
)


def _sc_deg_body(dst_hbm, zrow_hbm, one_hbm,
                 deg_out, dst0, dst1, one_v, deg_sh, sia, sib):
    c = lax.axis_index("c")
    s = lax.axis_index("s")
    wid = c * _NS + s
    ebase = wid * _EPT
    dsts = (dst0, dst1)
    si = (sia, sib)

    pltpu.sync_copy(zrow_hbm, deg_sh.at[pl.ds(s * _RPT, _RPT)])
    pltpu.sync_copy(one_hbm, one_v)
    pltpu.async_copy(dst_hbm.at[pl.ds(ebase, _BLK)], dst0, sia)
    plsc.subcore_barrier()

    def do_block(j, b):
        nb = 1 - b
        @pl.when(j + 1 < _NB)
        def _():
            pltpu.async_copy(
                dst_hbm.at[pl.ds(ebase + (j + 1) * _BLK, _BLK)],
                dsts[nb], si[nb])

        pltpu.make_async_copy(
            dst_hbm.at[pl.ds(ebase + j * _BLK, _BLK)], dsts[b], si[b]).wait()
        pltpu.sync_copy(one_v, deg_sh.at[dsts[b]], add=True)

    def step(i, carry):
        do_block(2 * i, 0)
        do_block(2 * i + 1, 1)
        return carry

    lax.fori_loop(0, _NB // 2, step, 0)
    if _NB % 2:
        do_block(_NB - 1, 0)

    plsc.subcore_barrier()
    base = c * _NPAD + s * _RPT
    pltpu.sync_copy(deg_sh.at[pl.ds(s * _RPT, _RPT)],
                    deg_out.at[pl.ds(base, _RPT)])


_sc_deg = pl.kernel(
    _sc_deg_body,
    out_type=jax.ShapeDtypeStruct((_NC * _NPAD, _D), jnp.float32),
    mesh=_sc_mesh,
    scratch_types=[
        pltpu.VMEM((_BLK,), jnp.int32),
        pltpu.VMEM((_BLK,), jnp.int32),
        pltpu.VMEM((_BLK, _D), jnp.float32),
        pltpu.VMEM_SHARED((_NPAD, _D), jnp.float32),
        pltpu.SemaphoreType.DMA,
        pltpu.SemaphoreType.DMA,
    ],
)


def _tc_layer0(p, degp, x, Wl, Wr, b, g, be):
    def body(p_ref, degp_ref, x_ref, wl_ref, wr_ref, b_ref, g_ref, be_ref,
             h_ref, dinv_ref):
        deg = degp_ref[0, :_N] + degp_ref[1, :_N]
        dinv = 1.0 / jnp.maximum(deg, 1.0)
        dinv_ref[...] = dinv
        agg = p_ref[0, :_N] + p_ref[1, :_N]
        mean = agg * dinv
        h = (jnp.dot(mean, wl_ref[...], preferred_element_type=jnp.float32)
             + jnp.dot(x_ref[...], wr_ref[...], preferred_element_type=jnp.float32)
             + b_ref[...])
        m = jnp.mean(h, axis=0, keepdims=True)
        hc = h - m
        v = jnp.mean(hc * hc, axis=0, keepdims=True)
        scale = g_ref[...] * lax.rsqrt(v + _EPS)
        h_ref[...] = jnp.maximum(hc * scale + be_ref[...], 0.0)

    return pl.pallas_call(
        body,
        out_shape=(jax.ShapeDtypeStruct((_N, _D), jnp.float32),
                   jax.ShapeDtypeStruct((_N, _D), jnp.float32)),
    )(p, degp, x, Wl, Wr, b, g, be)


def _tc_layer_mid(p, dinv, x, Wl, Wr, b, g, be):
    def body(p_ref, dinv_ref, x_ref, wl_ref, wr_ref, b_ref, g_ref, be_ref,
             h_ref):
        mean = (p_ref[0, :_N] + p_ref[1, :_N]) * dinv_ref[...]
        h = (jnp.dot(mean, wl_ref[...], preferred_element_type=jnp.float32)
             + jnp.dot(x_ref[...], wr_ref[...], preferred_element_type=jnp.float32)
             + b_ref[...])
        m = jnp.mean(h, axis=0, keepdims=True)
        hc = h - m
        v = jnp.mean(hc * hc, axis=0, keepdims=True)
        scale = g_ref[...] * lax.rsqrt(v + _EPS)
        h_ref[...] = jnp.maximum(hc * scale + be_ref[...], 0.0)

    return pl.pallas_call(
        body,
        out_shape=jax.ShapeDtypeStruct((_N, _D), jnp.float32),
    )(p, dinv, x, Wl, Wr, b, g, be)


def _tc_layer_last(p, dinv, x, Wl, Wr, b):
    def body(p_ref, dinv_ref, x_ref, wl_ref, wr_ref, b_ref, h_ref):
        mean = (p_ref[0, :_N] + p_ref[1, :_N]) * dinv_ref[...]
        h_ref[...] = (
            jnp.dot(mean, wl_ref[...], preferred_element_type=jnp.float32)
            + jnp.dot(x_ref[...], wr_ref[...], preferred_element_type=jnp.float32)
            + b_ref[...])

    return pl.pallas_call(
        body,
        out_shape=jax.ShapeDtypeStruct((_N, _D), jnp.float32),
    )(p, dinv, x, Wl, Wr, b)


def kernel(x, edge_index, Wl0, Wr0, b0, Wl1, Wr1, b1, Wl2, Wr2, b2,
           g0, be0, g1, be1):
    src = edge_index[0].astype(jnp.int32)
    dst = edge_index[1].astype(jnp.int32)
    zrow = jnp.zeros((_RPT, _D), jnp.float32)
    ones = jnp.ones((_BLK, _D), jnp.float32)

    b0r, b1r, b2r = b0.reshape(1, _D), b1.reshape(1, _D), b2.reshape(1, _D)
    g0r, g1r = g0.reshape(1, _D), g1.reshape(1, _D)
    be0r, be1r = be0.reshape(1, _D), be1.reshape(1, _D)

    degp = _sc_deg(dst, zrow, ones)
    p0 = _sc_agg(x, src, dst, zrow)
    h0, dinv = _tc_layer0(p0.reshape(_NC, _NPAD, _D),
                          degp.reshape(_NC, _NPAD, _D),
                          x, Wl0, Wr0, b0r, g0r, be0r)
    p1 = _sc_agg(h0, src, dst, zrow)
    h1 = _tc_layer_mid(p1.reshape(_NC, _NPAD, _D), dinv, h0,
                       Wl1, Wr1, b1r, g1r, be1r)
    p2 = _sc_agg(h1, src, dst, zrow)
    return _tc_layer_last(p2.reshape(_NC, _NPAD, _D), dinv, h1, Wl2, Wr2, b2r)
